# baseline (speedup 1.0000x reference)
import functools

import jax
import jax.numpy as jnp
from jax import lax
from jax.experimental import pallas as pl
from jax.experimental.pallas import tpu as pltpu
from jax.experimental.pallas import tpu_sc as plsc

_NUM_CORES = 2
_NUM_SUBCORES = 16
_NW = _NUM_CORES * _NUM_SUBCORES

_W = 512
_NBUF = 2


@functools.partial(jax.jit, static_argnums=(1, 2))
def _copy_cols(embT, d, n):
    n_full = n // _W
    tail = n - n_full * _W
    full = n_full // _NW
    rem = n_full - full * _NW
    B = min(_NBUF, full) or 1
    mesh = plsc.VectorSubcoreMesh(core_axis_name="c", subcore_axis_name="s")

    scratch = [pltpu.VMEM((d, _W), embT.dtype) for _ in range(B)]
    scratch += [pltpu.SemaphoreType.DMA for _ in range(2 * B)]
    if tail:
        scratch += [pltpu.VMEM((d, tail), embT.dtype),
                    pltpu.SemaphoreType.DMA, pltpu.SemaphoreType.DMA]

    @functools.partial(
        pl.kernel,
        mesh=mesh,
        out_type=jax.ShapeDtypeStruct((d, n), embT.dtype),
        scratch_types=scratch,
    )
    def body(emb_hbm, out_hbm, *refs):
        bufs = refs[:B]
        rsems = refs[B:2 * B]
        wsems = refs[2 * B:3 * B]
        wid = lax.axis_index("s") * _NUM_CORES + lax.axis_index("c")

        def col0(slot):
            return (wid + slot * _NW) * _W

        def start_read(slot):
            return pltpu.async_copy(
                emb_hbm.at[:, pl.ds(col0(slot), _W)],
                bufs[slot % B], rsems[slot % B])

        def start_write(slot):
            return pltpu.async_copy(
                bufs[slot % B],
                out_hbm.at[:, pl.ds(col0(slot), _W)],
                wsems[slot % B])

        S = full
        reads = [None] * S
        writes = [None] * S
        for j in range(min(B, S)):
            reads[j] = start_read(j)
        for i in range(S):
            k = i + B - 1
            if B > 1 and B <= k < S:
                writes[k - B].wait()
                reads[k] = start_read(k)
            reads[i].wait()
            writes[i] = start_write(i)
        pending = list(range(max(0, S - B), S))
        if rem:
            p = S % B
            if S - B >= 0:
                writes[S - B].wait()
                pending.remove(S - B)

            @pl.when(wid < rem)
            def _extra():
                pltpu.async_copy(
                    emb_hbm.at[:, pl.ds(col0(S), _W)],
                    bufs[p], rsems[p]).wait()
                pltpu.async_copy(
                    bufs[p], out_hbm.at[:, pl.ds(col0(S), _W)],
                    wsems[p]).wait()

        if tail:
            tbuf, trs, tws = refs[3 * B:3 * B + 3]
            c0 = n_full * _W

            @pl.when(wid == _NW - 1)
            def _tail():
                pltpu.async_copy(
                    emb_hbm.at[:, pl.ds(c0, tail)], tbuf, trs).wait()
                pltpu.async_copy(
                    tbuf, out_hbm.at[:, pl.ds(c0, tail)], tws).wait()

        for i in pending:
            writes[i].wait()

    return body(embT)


def kernel(emb, H):
    n, d = emb.shape
    del H
    return _copy_cols(emb.T, d, n).T


# speedup vs baseline: 1.0239x; 1.0239x over previous
import functools

import jax
import jax.numpy as jnp
from jax import lax
from jax.experimental import pallas as pl
from jax.experimental.pallas import tpu as pltpu
from jax.experimental.pallas import tpu_sc as plsc

_NUM_CORES = 2
_NUM_SUBCORES = 16
_NW = _NUM_CORES * _NUM_SUBCORES

_W = 128
_NBUF = 4


@functools.partial(jax.jit, static_argnums=(1, 2))
def _copy_cols(embT, d, n):
    n_full = n // _W
    tail = n - n_full * _W
    full = n_full // _NW
    rem = n_full - full * _NW
    B = min(_NBUF, full) or 1
    mesh = plsc.VectorSubcoreMesh(core_axis_name="c", subcore_axis_name="s")

    scratch = [pltpu.VMEM((d, _W), embT.dtype) for _ in range(B)]
    scratch += [pltpu.SemaphoreType.DMA for _ in range(2 * B)]
    if tail:
        scratch += [pltpu.VMEM((d, tail), embT.dtype),
                    pltpu.SemaphoreType.DMA, pltpu.SemaphoreType.DMA]

    @functools.partial(
        pl.kernel,
        mesh=mesh,
        out_type=jax.ShapeDtypeStruct((d, n), embT.dtype),
        scratch_types=scratch,
    )
    def body(emb_hbm, out_hbm, *refs):
        bufs = refs[:B]
        rsems = refs[B:2 * B]
        wsems = refs[2 * B:3 * B]
        wid = lax.axis_index("s") * _NUM_CORES + lax.axis_index("c")

        def col0(slot):
            return (wid + slot * _NW) * _W

        def start_read(slot):
            return pltpu.async_copy(
                emb_hbm.at[:, pl.ds(col0(slot), _W)],
                bufs[slot % B], rsems[slot % B])

        def start_write(slot):
            return pltpu.async_copy(
                bufs[slot % B],
                out_hbm.at[:, pl.ds(col0(slot), _W)],
                wsems[slot % B])

        S = full
        reads = [None] * S
        writes = [None] * S
        for j in range(min(B, S)):
            reads[j] = start_read(j)
        for i in range(S):
            k = i + B - 1
            if B > 1 and B <= k < S:
                writes[k - B].wait()
                reads[k] = start_read(k)
            reads[i].wait()
            writes[i] = start_write(i)
        pending = list(range(max(0, S - B), S))
        if rem:
            p = S % B
            if S - B >= 0:
                writes[S - B].wait()
                pending.remove(S - B)

            @pl.when(wid < rem)
            def _extra():
                pltpu.async_copy(
                    emb_hbm.at[:, pl.ds(col0(S), _W)],
                    bufs[p], rsems[p]).wait()
                pltpu.async_copy(
                    bufs[p], out_hbm.at[:, pl.ds(col0(S), _W)],
                    wsems[p]).wait()

        if tail:
            tbuf, trs, tws = refs[3 * B:3 * B + 3]
            c0 = n_full * _W

            @pl.when(wid == _NW - 1)
            def _tail():
                pltpu.async_copy(
                    emb_hbm.at[:, pl.ds(c0, tail)], tbuf, trs).wait()
                pltpu.async_copy(
                    tbuf, out_hbm.at[:, pl.ds(c0, tail)], tws).wait()

        for i in pending:
            writes[i].wait()

    return body(embT)


def kernel(emb, H):
    n, d = emb.shape
    del H
    return _copy_cols(emb.T, d, n).T
